# baseline (speedup 1.0000x reference)
import functools

import jax
import jax.numpy as jnp
from jax import lax
from jax.experimental import pallas as pl
from jax.experimental.pallas import tpu as pltpu
from jax.experimental.pallas import tpu_sc as plsc

NC = 2
NS = 16
NW = NC * NS
L = 16
W = 128


def _sc_tile_pos_embed(xt, embr, arf, gatef, n_tiles, n_tok, E, MM):
    TPW = n_tok // NW
    TAIL = n_tok - NW * TPW
    NCH = 8
    CHT = TPW // NCH
    NBI = 6
    assert TPW % NCH == 0 and TAIL <= CHT

    mesh = plsc.VectorSubcoreMesh(
        core_axis_name="c", subcore_axis_name="s",
        num_cores=NC, num_subcores=NS)

    @functools.partial(
        pl.kernel,
        out_type=jax.ShapeDtypeStruct((n_tok, 1, E), jnp.float32),
        mesh=mesh,
        scratch_types=[
            pltpu.VMEM((L,), jnp.int32),
            pltpu.VMEM((L,), jnp.float32),
            pltpu.VMEM((1, 1, E), jnp.float32),
        ] + [pltpu.VMEM((CHT, n_tiles, E), jnp.float32) for _ in range(NBI)]
          + [
            pltpu.SemaphoreType.DMA,
            pltpu.SemaphoreType.DMA,
            pltpu.SemaphoreType.DMA,
        ],
    )
    def k(x_hbm, emb_hbm, ar_hbm, g_hbm, out_hbm, arv, gv, ebuf,
          b0, b1, b2, b3, b4, b5, semx, seme, semo):
        wid = lax.axis_index("s") * NC + lax.axis_index("c")
        tok0 = wid * TPW
        bufs = (b0, b1, b2, b3, b4, b5)

        incopies = [None] * NCH
        for ch in range(min(NBI, NCH)):
            incopies[ch] = pltpu.async_copy(
                x_hbm.at[0, pl.ds(tok0 + ch * CHT, CHT), :, :],
                bufs[ch % NBI], semx)

        pltpu.sync_copy(ar_hbm, arv.at[pl.ds(0, 2)])
        pltpu.sync_copy(g_hbm, gv.at[pl.ds(0, 1)])
        var = arv[...]
        idx_s = var[0] * var[1] - 1
        tile = jnp.clip(idx_s, 0, n_tiles - 1)
        erow = jnp.clip(idx_s, 0, MM - 1)
        ecopy = pltpu.async_copy(emb_hbm.at[pl.ds(erow, 1), :, :], ebuf, seme)

        sx = jnp.full((L,), jnp.where(idx_s < n_tiles, 1.0, 0.0),
                      dtype=jnp.float32)
        g = jnp.full((L,), gv[...][0], dtype=jnp.float32)
        tg = 1.0 - 2.0 / (jnp.exp(g * 2.0) + 1.0)

        ecopy.wait()

        def scale_body(ci, carry):
            o = ci * L
            ebuf[0, 0, pl.ds(o, L)] = ebuf[0, 0, pl.ds(o, L)] * tg
            return carry

        lax.fori_loop(0, E // L, scale_body, 0)

        def compact(buf, nt):
            def ci_body(ci, carry):
                o = ci * L
                eg = ebuf[0, 0, pl.ds(o, L)]
                for t in range(nt):
                    buf[t, tile, pl.ds(o, L)] = (
                        buf[t, tile, pl.ds(o, L)] * sx + eg)
                return carry
            lax.fori_loop(0, E // L, ci_body, 0)

        outcopies = [None] * NCH
        for ch in range(NCH):
            incopies[ch].wait()
            compact(bufs[ch % NBI], CHT)
            outcopies[ch] = pltpu.async_copy(
                bufs[ch % NBI].at[pl.ds(0, CHT), tile, :],
                out_hbm.at[pl.ds(tok0 + ch * CHT, CHT), 0, :], semo)
            if ch + NBI < NCH:
                outcopies[ch].wait()
                incopies[ch + NBI] = pltpu.async_copy(
                    x_hbm.at[0, pl.ds(tok0 + (ch + NBI) * CHT, CHT), :, :],
                    bufs[ch % NBI], semx)

        if TAIL:
            @pl.when(wid == 0)
            def _():
                pltpu.async_copy(
                    x_hbm.at[0, pl.ds(NW * TPW, TAIL), :, :],
                    b0.at[pl.ds(0, TAIL), :, :], semx).wait()
                compact(b0, TAIL)
                pltpu.sync_copy(
                    b0.at[pl.ds(0, TAIL), tile, :],
                    out_hbm.at[pl.ds(NW * TPW, TAIL), 0, :])

        for ch in range(NCH):
            if ch + NBI >= NCH:
                outcopies[ch].wait()

    return k(xt, embr, arf, gatef)


def kernel(x, aspect_ratio, embedding, gate):
    bsz, n_tiles, n_tok, E = x.shape
    M = embedding.shape[0]
    xt = x.transpose(0, 2, 1, 3)
    embr = embedding.astype(jnp.float32).reshape(M * M, 1, E)
    arf = aspect_ratio.astype(jnp.int32).reshape(2)
    gatef = gate.astype(jnp.float32).reshape(1)
    out = _sc_tile_pos_embed(xt, embr, arf, gatef, n_tiles, n_tok, E, M * M)
    return out.reshape(bsz, 1, n_tok, E)


# speedup vs baseline: 1.0214x; 1.0214x over previous
import functools

import jax
import jax.numpy as jnp
from jax import lax
from jax.experimental import pallas as pl
from jax.experimental.pallas import tpu as pltpu
from jax.experimental.pallas import tpu_sc as plsc

NC = 2
NS = 16
NW = NC * NS
L = 16
W = 128


def _sc_tile_pos_embed(xt, embr, arf, gatef, n_tiles, n_tok, E, MM):
    TPW = n_tok // NW
    TAIL = n_tok - NW * TPW
    NCH = 4
    CHT = TPW // NCH
    NBI = 3
    assert TPW % NCH == 0 and TAIL <= CHT

    mesh = plsc.VectorSubcoreMesh(
        core_axis_name="c", subcore_axis_name="s",
        num_cores=NC, num_subcores=NS)

    @functools.partial(
        pl.kernel,
        out_type=jax.ShapeDtypeStruct((n_tok, 1, E), jnp.float32),
        mesh=mesh,
        scratch_types=[
            pltpu.VMEM((L,), jnp.int32),
            pltpu.VMEM((L,), jnp.float32),
            pltpu.VMEM((1, 1, E), jnp.float32),
        ] + [pltpu.VMEM((CHT, n_tiles, E), jnp.float32) for _ in range(NBI)]
          + [
            pltpu.SemaphoreType.DMA,
            pltpu.SemaphoreType.DMA,
            pltpu.SemaphoreType.DMA,
        ],
    )
    def k(x_hbm, emb_hbm, ar_hbm, g_hbm, out_hbm, arv, gv, ebuf,
          b0, b1, b2, semx, seme, semo):
        wid = lax.axis_index("s") * NC + lax.axis_index("c")
        tok0 = wid * TPW
        bufs = (b0, b1, b2)

        incopies = [None] * NCH
        for ch in range(min(NBI, NCH)):
            incopies[ch] = pltpu.async_copy(
                x_hbm.at[0, pl.ds(tok0 + ch * CHT, CHT), :, :],
                bufs[ch % NBI], semx)

        pltpu.sync_copy(ar_hbm, arv.at[pl.ds(0, 2)])
        pltpu.sync_copy(g_hbm, gv.at[pl.ds(0, 1)])
        var = arv[...]
        idx_s = var[0] * var[1] - 1
        tile = jnp.clip(idx_s, 0, n_tiles - 1)
        erow = jnp.clip(idx_s, 0, MM - 1)
        ecopy = pltpu.async_copy(emb_hbm.at[pl.ds(erow, 1), :, :], ebuf, seme)

        sx = jnp.full((L,), jnp.where(idx_s < n_tiles, 1.0, 0.0),
                      dtype=jnp.float32)
        g = jnp.full((L,), gv[...][0], dtype=jnp.float32)
        tg = 1.0 - 2.0 / (jnp.exp(g * 2.0) + 1.0)

        ecopy.wait()

        def scale_body(ci, carry):
            o = ci * L
            ebuf[0, 0, pl.ds(o, L)] = ebuf[0, 0, pl.ds(o, L)] * tg
            return carry

        lax.fori_loop(0, E // L, scale_body, 0)

        def compact(buf, nt):
            def ci_body(ci, carry):
                o = ci * L
                eg = ebuf[0, 0, pl.ds(o, L)]
                for t in range(nt):
                    buf[t, tile, pl.ds(o, L)] = (
                        buf[t, tile, pl.ds(o, L)] * sx + eg)
                return carry
            lax.fori_loop(0, E // L, ci_body, 0)

        outcopies = [None] * NCH
        for ch in range(NCH):
            incopies[ch].wait()
            compact(bufs[ch % NBI], CHT)
            outcopies[ch] = pltpu.async_copy(
                bufs[ch % NBI].at[pl.ds(0, CHT), tile, :],
                out_hbm.at[pl.ds(tok0 + ch * CHT, CHT), 0, :], semo)
            if ch + NBI < NCH:
                outcopies[ch].wait()
                incopies[ch + NBI] = pltpu.async_copy(
                    x_hbm.at[0, pl.ds(tok0 + (ch + NBI) * CHT, CHT), :, :],
                    bufs[ch % NBI], semx)

        if TAIL:
            @pl.when(wid == 0)
            def _():
                pltpu.async_copy(
                    x_hbm.at[0, pl.ds(NW * TPW, TAIL), :, :],
                    b0.at[pl.ds(0, TAIL), :, :], semx).wait()
                compact(b0, TAIL)
                pltpu.sync_copy(
                    b0.at[pl.ds(0, TAIL), tile, :],
                    out_hbm.at[pl.ds(NW * TPW, TAIL), 0, :])

        for ch in range(NCH):
            if ch + NBI >= NCH:
                outcopies[ch].wait()

    return k(xt, embr, arf, gatef)


def kernel(x, aspect_ratio, embedding, gate):
    bsz, n_tiles, n_tok, E = x.shape
    M = embedding.shape[0]
    xt = x.transpose(0, 2, 1, 3)
    embr = embedding.astype(jnp.float32).reshape(M * M, 1, E)
    arf = aspect_ratio.astype(jnp.int32).reshape(2)
    gatef = gate.astype(jnp.float32).reshape(1)
    out = _sc_tile_pos_embed(xt, embr, arf, gatef, n_tiles, n_tok, E, M * M)
    return out.reshape(bsz, 1, n_tok, E)
